# baseline (device time: 18930 ns/iter reference)
import functools

import jax
import jax.numpy as jnp
from jax import lax
from jax.experimental import pallas as pl
from jax.experimental.pallas import tpu as pltpu

N_DEV = 4


def kernel(x, dy):
    k_per, d = x.shape
    _, f = dy.shape
    m_per = d // N_DEV
    h = m_per // 2

    def body(x_ref, dy_ref, out_ref, dyb_ref, p_ref,
             h2ar_send, h2al_send,
             h1r_recv, h1l_recv, r2a_recv, r2b_recv, l2a_recv, l2b_recv,
             send_sems, recv_sems):
        my = lax.axis_index("i")
        left = lax.rem(my + N_DEV - 1, N_DEV)
        right = lax.rem(my + 1, N_DEV)
        c2 = lax.rem(my + 2, N_DEV)
        cr = right
        cl = left

        barrier_sem = pltpu.get_barrier_semaphore()
        for nbr in (left, right):
            pl.semaphore_signal(
                barrier_sem, inc=1,
                device_id=(nbr,), device_id_type=pl.DeviceIdType.MESH,
            )
        pl.semaphore_wait(barrier_sem, 2)

        dyb_ref[:, :] = dy_ref[:, :].astype(jnp.bfloat16)

        def pchunk(c):
            xs = x_ref[:, pl.ds(c * m_per, m_per)].astype(jnp.bfloat16)
            p_ref[pl.ds(c * m_per, m_per), :] = lax.dot_general(
                xs, dyb_ref[:, :],
                dimension_numbers=(((0,), (0,)), ((), ())),
                preferred_element_type=jnp.float32,
            ).astype(jnp.bfloat16)

        def rdma(src, dst, i, dev):
            return pltpu.make_async_remote_copy(
                src_ref=src, dst_ref=dst,
                send_sem=send_sems.at[i], recv_sem=recv_sems.at[i],
                device_id=(dev,), device_id_type=pl.DeviceIdType.MESH,
            )

        pchunk(c2)
        r1r = rdma(p_ref.at[pl.ds(c2 * m_per, h)], h1r_recv, 0, right)
        r1l = rdma(p_ref.at[pl.ds(c2 * m_per + h, h)], h1l_recv, 1, left)
        r1r.start()
        r1l.start()

        pchunk(cr)
        r2br = rdma(p_ref.at[pl.ds(cr * m_per + h, h)], r2b_recv, 2, right)
        r2br.start()
        pchunk(cl)
        r2bl = rdma(p_ref.at[pl.ds(cl * m_per, h)], l2b_recv, 3, left)
        r2bl.start()

        pchunk(my)

        r1r.wait_recv()
        h2ar_send[:, :] = h1r_recv[:, :] + p_ref[pl.ds(cr * m_per, h), :]
        r2ar = rdma(h2ar_send, r2a_recv, 4, right)
        r2ar.start()

        r1l.wait_recv()
        h2al_send[:, :] = h1l_recv[:, :] + p_ref[pl.ds(cl * m_per + h, h), :]
        r2al = rdma(h2al_send, l2a_recv, 5, left)
        r2al.start()

        r2ar.wait_recv()
        r2bl.wait_recv()
        out_ref[pl.ds(0, h), :] = (
            p_ref[pl.ds(my * m_per, h), :].astype(jnp.float32)
            + r2a_recv[:, :].astype(jnp.float32)
            + l2b_recv[:, :].astype(jnp.float32)
        )
        r2br.wait_recv()
        r2al.wait_recv()
        out_ref[pl.ds(h, h), :] = (
            p_ref[pl.ds(my * m_per + h, h), :].astype(jnp.float32)
            + r2b_recv[:, :].astype(jnp.float32)
            + l2a_recv[:, :].astype(jnp.float32)
        )

        for r in (r1r, r1l, r2br, r2bl, r2ar, r2al):
            r.wait_send()

        @functools.partial(pl.run_scoped, sem2=pltpu.SemaphoreType.REGULAR)
        def _(sem2):
            for nbr in (left, right):
                pl.semaphore_signal(
                    sem2, inc=1,
                    device_id=(nbr,), device_id_type=pl.DeviceIdType.MESH,
                )
            pl.semaphore_wait(sem2, 2)

    return pl.pallas_call(
        body,
        out_shape=jax.ShapeDtypeStruct((m_per, f), jnp.float32),
        in_specs=[
            pl.BlockSpec(memory_space=pltpu.VMEM),
            pl.BlockSpec(memory_space=pltpu.VMEM),
        ],
        out_specs=pl.BlockSpec(memory_space=pltpu.VMEM),
        scratch_shapes=[
            pltpu.VMEM((k_per, f), jnp.bfloat16),
            pltpu.VMEM((d, f), jnp.bfloat16),
            pltpu.VMEM((h, f), jnp.bfloat16),
            pltpu.VMEM((h, f), jnp.bfloat16),
            pltpu.VMEM((h, f), jnp.bfloat16),
            pltpu.VMEM((h, f), jnp.bfloat16),
            pltpu.VMEM((h, f), jnp.bfloat16),
            pltpu.VMEM((h, f), jnp.bfloat16),
            pltpu.VMEM((h, f), jnp.bfloat16),
            pltpu.VMEM((h, f), jnp.bfloat16),
            pltpu.SemaphoreType.DMA((6,)),
            pltpu.SemaphoreType.DMA((6,)),
        ],
        compiler_params=pltpu.CompilerParams(collective_id=0),
    )(x, dy)
